# Initial kernel scaffold; baseline (speedup 1.0000x reference)
import jax
import jax.numpy as jnp
from jax.experimental import pallas as pl


def kernel(group_embedding, user_embedding, item_embedding, members, rui_rows, rui_cols, rui_vals, rgu_rows, rgu_cols, rgu_vals, rgi_rows, rgi_cols, rgi_vals, Wg, bg, Wu, bu, Wi, bi):
    raise NotImplementedError("write your pallas kernel here")


# speedup vs baseline: 3.3201x; 3.3201x over previous
import functools
import jax
import jax.numpy as jnp
from jax import lax
from jax.experimental import pallas as pl
from jax.experimental.pallas import tpu as pltpu
from jax.experimental.pallas import tpu_sc as plsc

D = 64
NC = 2
NS = 16
CH = 128


def _ceil_to(x, m):
    return (x + m - 1) // m * m


@functools.lru_cache(maxsize=None)
def _make_segsum(nnz_pad, n_rows, h):
    del n_rows
    epp = nnz_pad // NS
    cps = epp // CH
    rps = h // NS
    zslices = (h + 128) // 128
    mesh = plsc.VectorSubcoreMesh(core_axis_name="c", subcore_axis_name="s")

    @functools.partial(
        pl.kernel,
        out_type=jax.ShapeDtypeStruct((2 * h, D), jnp.float32),
        mesh=mesh,
        scratch_types=[
            pltpu.VMEM((CH,), jnp.int32),
            pltpu.VMEM((CH,), jnp.float32),
            pltpu.VMEM((CH,), jnp.int32),
            pltpu.VMEM((CH, D), jnp.float32),
            pltpu.VMEM_SHARED((h + 128, D), jnp.float32),
            pltpu.SemaphoreType.DMA,
        ],
        compiler_params=pltpu.CompilerParams(use_tc_tiling_on_sc=False,
                                             needs_layout_passes=False),
    )
    def seg_kernel(scat_hbm, gath_hbm, vals_hbm, table_hbm, out_hbm,
                   gidx, valv, sidx, rows, shared, sem):
        cid = lax.axis_index("c")
        sid = lax.axis_index("s")
        base = cid * h
        col_ids = [lax.iota(jnp.int32, 16) + 16 * d4 for d4 in range(4)]

        def zrow(r, _):
            ev = lax.broadcast(r, (16,))
            for d4 in range(4):
                plsc.store_scatter(rows, [ev, col_ids[d4]],
                                   jnp.zeros((16,), jnp.float32))
            return _
        lax.fori_loop(0, CH, zrow, None)

        def zshared(i, _):
            s = sid + i * NS

            @pl.when(s < zslices)
            def _():
                pltpu.sync_copy(rows, shared.at[pl.ds(s * 128, 128)])
            return _
        lax.fori_loop(0, (zslices + NS - 1) // NS, zshared, None)
        plsc.subcore_barrier()

        eoff = sid * epp

        def chunk(j, _):
            pltpu.sync_copy(gath_hbm.at[pl.ds(eoff + j * CH, CH)], gidx)
            pltpu.sync_copy(vals_hbm.at[pl.ds(eoff + j * CH, CH)], valv)
            pltpu.sync_copy(scat_hbm.at[pl.ds(eoff + j * CH, CH)], sidx)
            pltpu.async_copy(table_hbm.at[gidx], rows, sem).wait()

            def scale(e, _):
                ev = lax.broadcast(e, (16,))
                vs = plsc.load_gather(valv, [ev])
                for d4 in range(4):
                    x = plsc.load_gather(rows, [ev, col_ids[d4]])
                    plsc.store_scatter(rows, [ev, col_ids[d4]], x * vs)
                return _
            lax.fori_loop(0, CH, scale, None)

            for g in range(CH // 16):
                loc = sidx[pl.ds(g * 16, 16)] - base
                ok = (loc >= 0) & (loc < h)
                sidx[pl.ds(g * 16, 16)] = jnp.where(ok, loc, h)
            pltpu.sync_copy(rows, shared.at[sidx], add=True)
            return _
        lax.fori_loop(0, cps, chunk, None)
        plsc.subcore_barrier()
        pltpu.sync_copy(shared.at[pl.ds(sid * rps, rps)],
                        out_hbm.at[pl.ds(base + sid * rps, rps)])

    return seg_kernel


def _segsum(scat_idx, gath_idx, vals, table, n_rows):
    nnz = scat_idx.shape[0]
    nnz_pad = _ceil_to(nnz, NS * CH)
    pad = nnz_pad - nnz
    if pad:
        scat_idx = jnp.concatenate([scat_idx, jnp.zeros((pad,), jnp.int32)])
        gath_idx = jnp.concatenate([gath_idx, jnp.zeros((pad,), jnp.int32)])
        vals = jnp.concatenate([vals, jnp.zeros((pad,), jnp.float32)])
    h = _ceil_to(_ceil_to(n_rows, 2) // 2, 128)
    k = _make_segsum(nnz_pad, n_rows, h)
    out = k(scat_idx, gath_idx, vals, table)
    return out[:n_rows]


@functools.lru_cache(maxsize=None)
def _make_gather(b_pad):
    bpw = b_pad // (NC * NS)
    mesh = plsc.VectorSubcoreMesh(core_axis_name="c", subcore_axis_name="s")

    @functools.partial(
        pl.kernel,
        out_type=jax.ShapeDtypeStruct((b_pad, D), jnp.float32),
        mesh=mesh,
        scratch_types=[
            pltpu.VMEM((bpw,), jnp.int32),
            pltpu.VMEM((bpw, D), jnp.float32),
            pltpu.SemaphoreType.DMA,
        ],
        compiler_params=pltpu.CompilerParams(use_tc_tiling_on_sc=False,
                                             needs_layout_passes=False),
    )
    def gather_kernel(table_hbm, idx_hbm, out_hbm, idx_v, rows_v, sem):
        wid = lax.axis_index("s") * NC + lax.axis_index("c")
        base = wid * bpw
        pltpu.sync_copy(idx_hbm.at[pl.ds(base, bpw)], idx_v)
        pltpu.async_copy(table_hbm.at[idx_v], rows_v, sem).wait()
        pltpu.sync_copy(rows_v, out_hbm.at[pl.ds(base, bpw)])

    return gather_kernel


def _gather_rows(table, idx):
    b = idx.shape[0]
    b_pad = _ceil_to(b, 8 * NC * NS)
    if b_pad != b:
        idx = jnp.concatenate([idx, jnp.zeros((b_pad - b,), jnp.int32)])
    return _make_gather(b_pad)(table, idx)


IB = 1024


def _att_colsum(item_pad, me_pad, n_items):
    ip, _ = item_pad.shape
    mp, _ = me_pad.shape
    nb = ip // IB

    def body(x_ref, me_ref, s_ref):
        @pl.when(pl.program_id(0) == 0)
        def _():
            s_ref[...] = jnp.zeros_like(s_ref)
        e = lax.dot_general(x_ref[...], me_ref[...], (((1,), (1,)), ((), ())),
                            preferred_element_type=jnp.float32)
        rid = pl.program_id(0) * IB + lax.broadcasted_iota(jnp.int32, (IB, 1), 0)
        contrib = jnp.where(rid < n_items, jnp.exp(e), 0.0)
        s_ref[...] += jnp.sum(contrib, axis=0, keepdims=True)

    return pl.pallas_call(
        body,
        grid=(nb,),
        in_specs=[pl.BlockSpec((IB, D), lambda i: (i, 0)),
                  pl.BlockSpec((mp, D), lambda i: (0, 0))],
        out_specs=pl.BlockSpec((1, mp), lambda i: (0, 0)),
        out_shape=jax.ShapeDtypeStruct((1, mp), jnp.float32),
    )(item_pad, me_pad)


def _att_apply(item_pad, me_pad, colsum):
    ip, _ = item_pad.shape
    mp, _ = me_pad.shape
    nb = ip // IB

    def body(x_ref, me_ref, s_ref, o_ref):
        x = x_ref[...]
        e = lax.dot_general(x, me_ref[...], (((1,), (1,)), ((), ())),
                            preferred_element_type=jnp.float32)
        w = jnp.exp(e) / s_ref[...]
        att = lax.dot_general(w, me_ref[...], (((1,), (0,)), ((), ())),
                              preferred_element_type=jnp.float32)
        o_ref[...] = att * x

    return pl.pallas_call(
        body,
        grid=(nb,),
        in_specs=[pl.BlockSpec((IB, D), lambda i: (i, 0)),
                  pl.BlockSpec((mp, D), lambda i: (0, 0)),
                  pl.BlockSpec((1, mp), lambda i: (0, 0))],
        out_specs=pl.BlockSpec((IB, D), lambda i: (i, 0)),
        out_shape=jax.ShapeDtypeStruct((ip, D), jnp.float32),
    )(item_pad, me_pad, colsum)


RB = 512


def _finish(acc):
    y = jnp.where(acc >= 0, acc, 0.01 * acc)
    nrm = jnp.sqrt(jnp.sum(y * y, axis=1, keepdims=True))
    return y / jnp.maximum(nrm, 1e-12)


def _dlin(x, w_ref, k):
    return lax.dot_general(x, w_ref[k], (((1,), (1,)), ((), ())),
                           preferred_element_type=jnp.float32)


def _combine_ui(x, a, b, W, bias, n_rows):
    np_ = _ceil_to(n_rows, RB)

    def padr(z):
        return jnp.pad(z, ((0, np_ - n_rows), (0, 0)))

    def body(x_ref, a_ref, b_ref, w_ref, bias_ref, o_ref):
        x = x_ref[...]
        a = a_ref[...]
        b = b_ref[...]
        acc = jnp.sum(bias_ref[...], axis=0, keepdims=True)
        acc = (acc + _dlin(x, w_ref, 0) + _dlin(a, w_ref, 1)
               + _dlin(a * x, w_ref, 2) + _dlin(b * x, w_ref, 3)
               + _dlin(b, w_ref, 4))
        o_ref[...] = _finish(acc)

    out = pl.pallas_call(
        body,
        grid=(np_ // RB,),
        in_specs=[pl.BlockSpec((RB, D), lambda i: (i, 0)),
                  pl.BlockSpec((RB, D), lambda i: (i, 0)),
                  pl.BlockSpec((RB, D), lambda i: (i, 0)),
                  pl.BlockSpec((5, D, D), lambda i: (0, 0, 0)),
                  pl.BlockSpec((5, D), lambda i: (0, 0))],
        out_specs=pl.BlockSpec((RB, D), lambda i: (i, 0)),
        out_shape=jax.ShapeDtypeStruct((np_, D), jnp.float32),
    )(padr(x), padr(a), padr(b), W, bias)
    return out[:n_rows]


def _combine_g(g, a, b, c, W, bias, n_rows):
    np_ = _ceil_to(n_rows, RB)

    def padr(z):
        return jnp.pad(z, ((0, np_ - n_rows), (0, 0)))

    def body(g_ref, a_ref, b_ref, c_ref, w_ref, bias_ref, o_ref):
        g = g_ref[...]
        a = a_ref[...]
        b = b_ref[...]
        c = c_ref[...]
        acc = jnp.sum(bias_ref[...], axis=0, keepdims=True)
        acc = (acc + _dlin(g, w_ref, 0) + _dlin(a, w_ref, 1)
               + _dlin(b * g, w_ref, 2) + _dlin(a * g, w_ref, 3)
               + _dlin(c, w_ref, 4))
        o_ref[...] = _finish(acc)

    out = pl.pallas_call(
        body,
        grid=(np_ // RB,),
        in_specs=[pl.BlockSpec((RB, D), lambda i: (i, 0)),
                  pl.BlockSpec((RB, D), lambda i: (i, 0)),
                  pl.BlockSpec((RB, D), lambda i: (i, 0)),
                  pl.BlockSpec((RB, D), lambda i: (i, 0)),
                  pl.BlockSpec((5, D, D), lambda i: (0, 0, 0)),
                  pl.BlockSpec((5, D), lambda i: (0, 0))],
        out_specs=pl.BlockSpec((RB, D), lambda i: (i, 0)),
        out_shape=jax.ShapeDtypeStruct((np_, D), jnp.float32),
    )(padr(g), padr(a), padr(b), padr(c), W, bias)
    return out[:n_rows]


def kernel(group_embedding, user_embedding, item_embedding, members,
           rui_rows, rui_cols, rui_vals, rgu_rows, rgu_cols, rgu_vals,
           rgi_rows, rgi_cols, rgi_vals, Wg, bg, Wu, bu, Wi, bi):
    G, U, I = group_embedding.shape[0], user_embedding.shape[0], item_embedding.shape[0]

    rui_ei = _segsum(rui_rows, rui_cols, rui_vals, item_embedding, U)
    rgu_t_eg = _segsum(rgu_cols, rgu_rows, rgu_vals, group_embedding, U)
    rui_t_eu = _segsum(rui_cols, rui_rows, rui_vals, user_embedding, I)
    rgi_t_eg = _segsum(rgi_cols, rgi_rows, rgi_vals, group_embedding, I)
    rgi_ei = _segsum(rgi_rows, rgi_cols, rgi_vals, item_embedding, G)
    rgu_eu = _segsum(rgu_rows, rgu_cols, rgu_vals, user_embedding, G)

    mflat = members.reshape(-1).astype(jnp.int32)
    n_mem = mflat.shape[0]
    me = _gather_rows(user_embedding, mflat)
    mp = _ceil_to(n_mem, 8 * NC * NS)
    me_pad = jnp.where(
        (jnp.arange(mp) < n_mem)[:, None], me, 0.0)
    ip = _ceil_to(I, IB)
    item_pad = jnp.pad(item_embedding, ((0, ip - I), (0, 0)))
    colsum = _att_colsum(item_pad, me_pad, I)
    attentive = _att_apply(item_pad, me_pad, colsum)

    atten_g = _segsum(rgi_rows, rgi_cols, rgi_vals, attentive, G)

    nu = _combine_ui(user_embedding, rui_ei, rgu_t_eg, Wu, bu, U)
    ni = _combine_ui(item_embedding, rui_t_eu, rgi_t_eg, Wi, bi, I)
    ng = _combine_g(group_embedding, rgi_ei, rgu_eu, atten_g, Wg, bg, G)
    return ng, nu, ni


# speedup vs baseline: 4.6567x; 1.4026x over previous
import functools
import jax
import jax.numpy as jnp
from jax import lax
from jax.experimental import pallas as pl
from jax.experimental.pallas import tpu as pltpu
from jax.experimental.pallas import tpu_sc as plsc

D = 64
NC = 2
NS = 16
CH = 128


def _ceil_to(x, m):
    return (x + m - 1) // m * m


@functools.lru_cache(maxsize=None)
def _make_segsum(nnz_pad, h, edge_part):
    nworkers = NC * NS if edge_part else NS
    epp = nnz_pad // nworkers
    cps = epp // CH
    rps = h // NS
    zslices = (h + 128) // 128
    mesh = plsc.VectorSubcoreMesh(core_axis_name="c", subcore_axis_name="s")
    out_sds = (jax.ShapeDtypeStruct((NC, h, D), jnp.float32) if edge_part
               else jax.ShapeDtypeStruct((2 * h, D), jnp.float32))

    @functools.partial(
        pl.kernel,
        out_type=out_sds,
        mesh=mesh,
        scratch_types=[
            pltpu.VMEM((CH,), jnp.int32), pltpu.VMEM((CH,), jnp.int32),
            pltpu.VMEM((CH,), jnp.float32), pltpu.VMEM((CH,), jnp.float32),
            pltpu.VMEM((CH,), jnp.int32), pltpu.VMEM((CH,), jnp.int32),
            pltpu.VMEM((CH, D), jnp.float32), pltpu.VMEM((CH, D), jnp.float32),
            pltpu.VMEM_SHARED((h + 128, D), jnp.float32),
            pltpu.SemaphoreType.DMA, pltpu.SemaphoreType.DMA,
            pltpu.SemaphoreType.DMA, pltpu.SemaphoreType.DMA,
            pltpu.SemaphoreType.DMA, pltpu.SemaphoreType.DMA,
        ],
        compiler_params=pltpu.CompilerParams(use_tc_tiling_on_sc=False,
                                             needs_layout_passes=False),
    )
    def seg_kernel(scat_hbm, gath_hbm, vals_hbm, table_hbm, out_hbm,
                   gi0, gi1, vv0, vv1, si0, si1, ro0, ro1, shared,
                   semi0, semi1, semg0, semg1, sems0, sems1):
        cid = lax.axis_index("c")
        sid = lax.axis_index("s")
        base = 0 if edge_part else cid * h
        col_ids = [lax.iota(jnp.int32, 16) + 16 * d4 for d4 in range(4)]
        gis, vvs, sis, ros = (gi0, gi1), (vv0, vv1), (si0, si1), (ro0, ro1)
        semis, semgs, semss = (semi0, semi1), (semg0, semg1), (sems0, sems1)

        def zrow(r, _):
            ev = lax.broadcast(r, (16,))
            for d4 in range(4):
                plsc.store_scatter(ro0, [ev, col_ids[d4]],
                                   jnp.zeros((16,), jnp.float32))
            return _
        lax.fori_loop(0, CH, zrow, None)

        def zshared(i, _):
            s = sid + i * NS

            @pl.when(s < zslices)
            def _():
                pltpu.sync_copy(ro0, shared.at[pl.ds(s * 128, 128)])
            return _
        lax.fori_loop(0, (zslices + NS - 1) // NS, zshared, None)
        plsc.subcore_barrier()

        eoff = ((cid * NS + sid) if edge_part else sid) * epp

        def issue_idx(j, b):
            off = eoff + j * CH
            pltpu.async_copy(gath_hbm.at[pl.ds(off, CH)], gis[b], semis[b])
            pltpu.async_copy(vals_hbm.at[pl.ds(off, CH)], vvs[b], semis[b])
            pltpu.async_copy(scat_hbm.at[pl.ds(off, CH)], sis[b], semis[b])

        def drain_idx(b):
            pltpu.make_async_copy(gath_hbm.at[pl.ds(0, CH)], gis[b], semis[b]).wait()
            pltpu.make_async_copy(vals_hbm.at[pl.ds(0, CH)], vvs[b], semis[b]).wait()
            pltpu.make_async_copy(scat_hbm.at[pl.ds(0, CH)], sis[b], semis[b]).wait()

        def drain_scat(b):
            pltpu.make_async_copy(ros[b], shared.at[sis[b]], semss[b]).wait()

        def process(b):
            pltpu.make_async_copy(table_hbm.at[gis[b]], ros[b], semgs[b]).wait()

            def scale(e, _):
                ev = lax.broadcast(e, (16,))
                vs = plsc.load_gather(vvs[b], [ev])
                for d4 in range(4):
                    x = plsc.load_gather(ros[b], [ev, col_ids[d4]])
                    plsc.store_scatter(ros[b], [ev, col_ids[d4]], x * vs)
                return _
            lax.fori_loop(0, CH, scale, None)
            for g in range(CH // 16):
                loc = sis[b][pl.ds(g * 16, 16)] - base
                ok = (loc >= 0) & (loc < h)
                sis[b][pl.ds(g * 16, 16)] = jnp.where(ok, loc, h)
            pltpu.async_copy(ros[b], shared.at[sis[b]], semss[b], add=True)

        issue_idx(0, 0)
        issue_idx(1, 1)

        def pair(p, _):
            drain_idx(0)

            @pl.when(p > 0)
            def _():
                drain_scat(0)
            pltpu.async_copy(table_hbm.at[gis[0]], ros[0], semgs[0])
            drain_idx(1)

            @pl.when(p > 0)
            def _():
                drain_scat(1)
            pltpu.async_copy(table_hbm.at[gis[1]], ros[1], semgs[1])
            process(0)

            @pl.when(2 * p + 2 < cps)
            def _():
                issue_idx(2 * p + 2, 0)
            process(1)

            @pl.when(2 * p + 3 < cps)
            def _():
                issue_idx(2 * p + 3, 1)
            return _
        lax.fori_loop(0, cps // 2, pair, None)
        drain_scat(0)
        drain_scat(1)
        plsc.subcore_barrier()
        if edge_part:
            pltpu.sync_copy(shared.at[pl.ds(sid * rps, rps)],
                            out_hbm.at[cid, pl.ds(sid * rps, rps)])
        else:
            pltpu.sync_copy(shared.at[pl.ds(sid * rps, rps)],
                            out_hbm.at[pl.ds(cid * h + sid * rps, rps)])

    return seg_kernel


def _segsum(scat_idx, gath_idx, vals, table, n_rows):
    edge_part = n_rows <= 12288
    nnz = scat_idx.shape[0]
    nnz_pad = _ceil_to(nnz, 2 * CH * (NC * NS if edge_part else NS))
    pad = nnz_pad - nnz
    if pad:
        scat_idx = jnp.concatenate([scat_idx, jnp.zeros((pad,), jnp.int32)])
        gath_idx = jnp.concatenate([gath_idx, jnp.zeros((pad,), jnp.int32)])
        vals = jnp.concatenate([vals, jnp.zeros((pad,), jnp.float32)])
    if edge_part:
        h = _ceil_to(n_rows, 128)
        out = _make_segsum(nnz_pad, h, True)(scat_idx, gath_idx, vals, table)
        return [out[0, :n_rows], out[1, :n_rows]]
    h = _ceil_to(_ceil_to(n_rows, 2) // 2, 128)
    out = _make_segsum(nnz_pad, h, False)(scat_idx, gath_idx, vals, table)
    return [out[:n_rows]]


@functools.lru_cache(maxsize=None)
def _make_gather(b_pad):
    bpw = b_pad // (NC * NS)
    mesh = plsc.VectorSubcoreMesh(core_axis_name="c", subcore_axis_name="s")

    @functools.partial(
        pl.kernel,
        out_type=jax.ShapeDtypeStruct((b_pad, D), jnp.float32),
        mesh=mesh,
        scratch_types=[
            pltpu.VMEM((bpw,), jnp.int32),
            pltpu.VMEM((bpw, D), jnp.float32),
            pltpu.SemaphoreType.DMA,
        ],
        compiler_params=pltpu.CompilerParams(use_tc_tiling_on_sc=False,
                                             needs_layout_passes=False),
    )
    def gather_kernel(table_hbm, idx_hbm, out_hbm, idx_v, rows_v, sem):
        wid = lax.axis_index("s") * NC + lax.axis_index("c")
        base = wid * bpw
        pltpu.sync_copy(idx_hbm.at[pl.ds(base, bpw)], idx_v)
        pltpu.async_copy(table_hbm.at[idx_v], rows_v, sem).wait()
        pltpu.sync_copy(rows_v, out_hbm.at[pl.ds(base, bpw)])

    return gather_kernel


def _gather_rows(table, idx):
    b = idx.shape[0]
    b_pad = _ceil_to(b, 8 * NC * NS)
    if b_pad != b:
        idx = jnp.concatenate([idx, jnp.zeros((b_pad - b,), jnp.int32)])
    return _make_gather(b_pad)(table, idx)


IB = 1024


def _att_colsum(item_pad, me_pad, n_items):
    ip, _ = item_pad.shape
    mp, _ = me_pad.shape
    nb = ip // IB

    def body(x_ref, me_ref, s_ref):
        @pl.when(pl.program_id(0) == 0)
        def _():
            s_ref[...] = jnp.zeros_like(s_ref)
        e = lax.dot_general(x_ref[...], me_ref[...], (((1,), (1,)), ((), ())),
                            preferred_element_type=jnp.float32)
        rid = pl.program_id(0) * IB + lax.broadcasted_iota(jnp.int32, (IB, 1), 0)
        contrib = jnp.where(rid < n_items, jnp.exp(e), 0.0)
        s_ref[...] += jnp.sum(contrib, axis=0, keepdims=True)

    return pl.pallas_call(
        body,
        grid=(nb,),
        in_specs=[pl.BlockSpec((IB, D), lambda i: (i, 0)),
                  pl.BlockSpec((mp, D), lambda i: (0, 0))],
        out_specs=pl.BlockSpec((1, mp), lambda i: (0, 0)),
        out_shape=jax.ShapeDtypeStruct((1, mp), jnp.float32),
    )(item_pad, me_pad)


def _att_apply(item_pad, me_pad, colsum):
    ip, _ = item_pad.shape
    mp, _ = me_pad.shape
    nb = ip // IB

    def body(x_ref, me_ref, s_ref, o_ref):
        x = x_ref[...]
        e = lax.dot_general(x, me_ref[...], (((1,), (1,)), ((), ())),
                            preferred_element_type=jnp.float32)
        w = jnp.exp(e) / s_ref[...]
        att = lax.dot_general(w, me_ref[...], (((1,), (0,)), ((), ())),
                              preferred_element_type=jnp.float32)
        o_ref[...] = att * x

    return pl.pallas_call(
        body,
        grid=(nb,),
        in_specs=[pl.BlockSpec((IB, D), lambda i: (i, 0)),
                  pl.BlockSpec((mp, D), lambda i: (0, 0)),
                  pl.BlockSpec((1, mp), lambda i: (0, 0))],
        out_specs=pl.BlockSpec((IB, D), lambda i: (i, 0)),
        out_shape=jax.ShapeDtypeStruct((ip, D), jnp.float32),
    )(item_pad, me_pad, colsum)


RB = 512


def _finish(acc):
    y = jnp.where(acc >= 0, acc, 0.01 * acc)
    nrm = jnp.sqrt(jnp.sum(y * y, axis=1, keepdims=True))
    return y / jnp.maximum(nrm, 1e-12)


def _dlin(x, w_ref, k):
    return lax.dot_general(x, w_ref[k], (((1,), (1,)), ((), ())),
                           preferred_element_type=jnp.float32)


def _combine(base, part_lists, W, bias, n_rows, group_pattern):
    np_ = _ceil_to(n_rows, RB)

    def padr(z):
        return jnp.pad(z, ((0, np_ - n_rows), (0, 0)))

    counts = [len(pl_) for pl_ in part_lists]

    def body(*refs):
        x = refs[0][...]
        pos = 1
        aggs = []
        for c in counts:
            agg = refs[pos][...]
            for r in refs[pos + 1:pos + c]:
                agg = agg + r[...]
            aggs.append(agg)
            pos += c
        w_ref, bias_ref, o_ref = refs[pos], refs[pos + 1], refs[pos + 2]
        acc = jnp.sum(bias_ref[...], axis=0, keepdims=True)
        if group_pattern:
            a, b, c = aggs
            acc = (acc + _dlin(x, w_ref, 0) + _dlin(a, w_ref, 1)
                   + _dlin(b * x, w_ref, 2) + _dlin(a * x, w_ref, 3)
                   + _dlin(c, w_ref, 4))
        else:
            a, b = aggs
            acc = (acc + _dlin(x, w_ref, 0) + _dlin(a, w_ref, 1)
                   + _dlin(a * x, w_ref, 2) + _dlin(b * x, w_ref, 3)
                   + _dlin(b, w_ref, 4))
        o_ref[...] = _finish(acc)

    flat_parts = [p for pl_ in part_lists for p in pl_]
    n_data = 1 + len(flat_parts)
    out = pl.pallas_call(
        body,
        grid=(np_ // RB,),
        in_specs=[pl.BlockSpec((RB, D), lambda i: (i, 0))] * n_data
        + [pl.BlockSpec((5, D, D), lambda i: (0, 0, 0)),
           pl.BlockSpec((5, D), lambda i: (0, 0))],
        out_specs=pl.BlockSpec((RB, D), lambda i: (i, 0)),
        out_shape=jax.ShapeDtypeStruct((np_, D), jnp.float32),
    )(padr(base), *[padr(p) for p in flat_parts], W, bias)
    return out[:n_rows]


def kernel(group_embedding, user_embedding, item_embedding, members,
           rui_rows, rui_cols, rui_vals, rgu_rows, rgu_cols, rgu_vals,
           rgi_rows, rgi_cols, rgi_vals, Wg, bg, Wu, bu, Wi, bi):
    G, U, I = group_embedding.shape[0], user_embedding.shape[0], item_embedding.shape[0]

    rui_ei = _segsum(rui_rows, rui_cols, rui_vals, item_embedding, U)
    rgu_t_eg = _segsum(rgu_cols, rgu_rows, rgu_vals, group_embedding, U)
    rui_t_eu = _segsum(rui_cols, rui_rows, rui_vals, user_embedding, I)
    rgi_t_eg = _segsum(rgi_cols, rgi_rows, rgi_vals, group_embedding, I)
    rgi_ei = _segsum(rgi_rows, rgi_cols, rgi_vals, item_embedding, G)
    rgu_eu = _segsum(rgu_rows, rgu_cols, rgu_vals, user_embedding, G)

    mflat = members.reshape(-1).astype(jnp.int32)
    n_mem = mflat.shape[0]
    me = _gather_rows(user_embedding, mflat)
    mp = _ceil_to(n_mem, 8 * NC * NS)
    me_pad = jnp.where(
        (jnp.arange(mp) < n_mem)[:, None], me, 0.0)
    ip = _ceil_to(I, IB)
    item_pad = jnp.pad(item_embedding, ((0, ip - I), (0, 0)))
    colsum = _att_colsum(item_pad, me_pad, I)
    attentive = _att_apply(item_pad, me_pad, colsum)

    atten_g = _segsum(rgi_rows, rgi_cols, rgi_vals, attentive, G)

    nu = _combine(user_embedding, [rui_ei, rgu_t_eg], Wu, bu, U, False)
    ni = _combine(item_embedding, [rui_t_eu, rgi_t_eg], Wi, bi, I, False)
    ng = _combine(group_embedding, [rgi_ei, rgu_eu, atten_g], Wg, bg, G, True)
    return ng, nu, ni
